# baseline (speedup 1.0000x reference)
import functools

import jax
import jax.numpy as jnp
from jax import lax
from jax.experimental import pallas as pl
from jax.experimental.pallas import tpu as pltpu
from jax.experimental.pallas import tpu_sc as plsc

N_NODES = 10000
N_EDGES = 320000
D = 128
DH = 64
D_BOND = 16
N_BOND_PAD = 8
DEPTH_G = 5

NBLK = 25
BN = N_NODES // NBLK
EROWS = N_EDGES // D

NW = 32
CH = 96
NCH = 106
EPW = NCH * CH
EPAD = NW * EPW
ERP = EPAD // D
NPAD = 10112
RPT = NPAD // 16

NBUF = 2
NSUPER = NCH // NBUF


def _eidx_body(src_ref, bt_ref, eidx_ref):
    eidx_ref[...] = src_ref[...] + bt_ref[...] * N_NODES


def _eidx(src2d, bt2d):
    return pl.pallas_call(
        _eidx_body,
        out_shape=jax.ShapeDtypeStruct((ERP, D), jnp.int32),
    )(src2d, bt2d)


def _prep_body(fnode_ref, ea_ref, wa_ref, wo1a_ref,
               eb_ref, wb_ref, bin_ref,
               p_ref, fo_ref, t0_ref):
    fn = fnode_ref[0, 0, :]
    onehot = (fn[:, None] == lax.broadcasted_iota(jnp.int32, (BN, D), 1))
    fatom = jnp.dot(onehot.astype(jnp.float32), ea_ref[...],
                    preferred_element_type=jnp.float32)
    pblk = jnp.dot(fatom, wa_ref[...], preferred_element_type=jnp.float32)
    p_ref[...] = pblk
    fo_ref[...] = jnp.dot(fatom, wo1a_ref[...],
                          preferred_element_type=jnp.float32)
    bv = jnp.dot(eb_ref[...], wb_ref[...],
                 preferred_element_type=jnp.float32) + bin_ref[...]
    for b in range(4):
        t0_ref[b] = jnp.maximum(pblk + bv[b:b + 1, :], 0.0)


def _prep(fnode3, ea_pad, w_a, w_o1a, eb_pad, w_b, b_in2):
    full = lambda shp: pl.BlockSpec(shp, lambda i: tuple(0 for _ in shp))
    return pl.pallas_call(
        _prep_body,
        grid=(NBLK,),
        in_specs=[
            pl.BlockSpec((1, 1, BN), lambda i: (i, 0, 0)),
            full((D, D)), full((D, D)), full((D, D)),
            full((N_BOND_PAD, D_BOND)), full((D_BOND, D)), full((1, D)),
        ],
        out_specs=[
            pl.BlockSpec((BN, D), lambda i: (i, 0)),
            pl.BlockSpec((BN, D), lambda i: (i, 0)),
            pl.BlockSpec((4, BN, D), lambda i: (0, i, 0)),
        ],
        out_shape=[
            jax.ShapeDtypeStruct((N_NODES, D), jnp.float32),
            jax.ShapeDtypeStruct((N_NODES, D), jnp.float32),
            jax.ShapeDtypeStruct((4, N_NODES, D), jnp.float32),
        ],
    )(fnode3, ea_pad, w_a, w_o1a, eb_pad, w_b, b_in2)


def _combine_body(part_ref, p_ref, wh_ref, bh_ref, eb_ref, wb_ref, bin_ref,
                  t_ref):
    agg = part_ref[0] + part_ref[1]
    q = p_ref[...] + jnp.dot(agg, wh_ref[...],
                             preferred_element_type=jnp.float32) + bh_ref[...]
    bv = jnp.dot(eb_ref[...], wb_ref[...],
                 preferred_element_type=jnp.float32) + bin_ref[...]
    for b in range(4):
        t_ref[b] = jnp.maximum(q + bv[b:b + 1, :], 0.0)


def _combine(part, p, w_h, b_h2, eb_pad, w_b, b_in2):
    full = lambda shp: pl.BlockSpec(shp, lambda i: tuple(0 for _ in shp))
    return pl.pallas_call(
        _combine_body,
        grid=(NBLK,),
        in_specs=[
            pl.BlockSpec((2, BN, D), lambda i: (0, i, 0)),
            pl.BlockSpec((BN, D), lambda i: (i, 0)),
            full((D, D)), full((1, D)),
            full((N_BOND_PAD, D_BOND)), full((D_BOND, D)), full((1, D)),
        ],
        out_specs=pl.BlockSpec((4, BN, D), lambda i: (0, i, 0)),
        out_shape=jax.ShapeDtypeStruct((4, N_NODES, D), jnp.float32),
    )(part, p, w_h, b_h2, eb_pad, w_b, b_in2)


def _head_body(part_ref, fo_ref, wo1b_ref, bo1_ref, wo2_ref, bo2_ref,
               out_ref):
    a = part_ref[0] + part_ref[1]
    h = jnp.maximum(
        fo_ref[...] + jnp.dot(a, wo1b_ref[...],
                              preferred_element_type=jnp.float32)
        + bo1_ref[...], 0.0)
    out_ref[...] = jnp.dot(h, wo2_ref[...],
                           preferred_element_type=jnp.float32) + bo2_ref[...]


def _head(part, fo, w_o1b, b_o12, w_o2, b_o22):
    full = lambda shp: pl.BlockSpec(shp, lambda i: tuple(0 for _ in shp))
    return pl.pallas_call(
        _head_body,
        grid=(NBLK,),
        in_specs=[
            pl.BlockSpec((2, BN, D), lambda i: (0, i, 0)),
            pl.BlockSpec((BN, D), lambda i: (i, 0)),
            full((D, D)), full((1, D)), full((D, D)), full((1, D)),
        ],
        out_specs=pl.BlockSpec((BN, D), lambda i: (i, 0)),
        out_shape=jax.ShapeDtypeStruct((N_NODES, D), jnp.float32),
    )(part, fo, w_o1b, b_o12, w_o2, b_o22)


def _sc_body(t_hbm, eidx_hbm, dst_hbm, zeros_hbm, out_hbm,
             agg_s, eidx_v, dst_v, rows_v, *sems):
    sg = sems[:NBUF]
    ss_a, ss_b = sems[NBUF], sems[NBUF + 1]
    cid = lax.axis_index("c")
    sid = lax.axis_index("s")
    w = sid * 2 + cid

    def fire_gather(c, b):
        base = pl.multiple_of(c * CH, 8)
        pltpu.async_copy(t_hbm.at[eidx_v.at[pl.ds(base, CH)]],
                         rows_v.at[b], sg[b])

    def wait_gather(b):
        pltpu.make_async_copy(t_hbm.at[pl.ds(0, CH)], rows_v.at[b],
                              sg[b]).wait()

    def fire_scatter(c, b, ss):
        pltpu.async_copy(rows_v.at[b], agg_s.at[dst_v.at[c]], ss, add=True)

    def drain(ss):
        pltpu.make_async_copy(t_hbm.at[pl.ds(0, CH)], rows_v.at[0],
                              ss).wait()

    pltpu.sync_copy(zeros_hbm, agg_s.at[pl.ds(sid * RPT, RPT)])
    pltpu.sync_copy(eidx_hbm.at[w], eidx_v)
    pltpu.sync_copy(dst_hbm.at[w], dst_v)
    plsc.subcore_barrier()

    fire_gather(0, 0)

    def super_block(s_idx, carry):
        c0 = s_idx * NBUF

        @pl.when(s_idx > 0)
        def _():
            drain(ss_b)
        fire_gather(c0 + 1, 1)
        wait_gather(0)
        fire_scatter(c0, 0, ss_a)

        @pl.when(s_idx < NSUPER - 1)
        def _():
            drain(ss_a)
            fire_gather(c0 + 2, 0)
        wait_gather(1)
        fire_scatter(c0 + 1, 1, ss_b)
        return carry

    lax.fori_loop(0, NSUPER, super_block, 0)
    drain(ss_a)
    drain(ss_b)

    plsc.subcore_barrier()
    pltpu.sync_copy(agg_s.at[pl.ds(sid * RPT, RPT)],
                    out_hbm.at[cid, pl.ds(sid * RPT, RPT)])


_sc_pass = functools.partial(
    pl.kernel,
    out_type=jax.ShapeDtypeStruct((2, NPAD, D), jnp.float32),
    mesh=plsc.VectorSubcoreMesh(core_axis_name="c", subcore_axis_name="s"),
    scratch_types=[
        pltpu.VMEM_SHARED((NPAD, D), jnp.float32),
        pltpu.VMEM((EPW,), jnp.int32),
        pltpu.VMEM((NCH, CH), jnp.int32),
        pltpu.VMEM((NBUF, CH, D), jnp.float32),
    ] + [pltpu.SemaphoreType.DMA] * (NBUF + 2),
)(_sc_body)


def kernel(fnode, edge_index, bond_type, E_a, E_b,
           W_in, b_in, W_h, b_h, W_o1, b_o1, W_o2, b_o2):
    fnode3 = fnode.astype(jnp.int32).reshape(NBLK, 1, BN)
    npe = EPAD - N_EDGES
    pad0 = jnp.zeros((npe,), jnp.int32)
    src_p = jnp.concatenate([edge_index[0].astype(jnp.int32), pad0])
    bt_p = jnp.concatenate([bond_type.astype(jnp.int32), pad0])
    dst_p = jnp.concatenate([edge_index[1].astype(jnp.int32),
                             jnp.full((npe,), N_NODES, jnp.int32)])
    src2d = src_p.reshape(ERP, D)
    bt2d = bt_p.reshape(ERP, D)
    dst3 = dst_p.reshape(NW, NCH, CH)

    ea_pad = jnp.zeros((D, D), jnp.float32).at[:E_a.shape[0]].set(E_a)
    eb_pad = jnp.zeros((N_BOND_PAD, D_BOND), jnp.float32).at[:4].set(E_b)
    w_a, w_b = W_in[:D], W_in[D:]
    w_o1a, w_o1b = W_o1[:D], W_o1[D:]
    b_in2 = b_in.reshape(1, D)
    b_h2 = b_h.reshape(1, D)
    b_o12 = b_o1.reshape(1, D)
    b_o22 = b_o2.reshape(1, D)
    zeros = jnp.zeros((RPT, D), jnp.float32)

    p, fo, t0 = _prep(fnode3, ea_pad, w_a, w_o1a, eb_pad, w_b, b_in2)
    eidx_w = _eidx(src2d, bt2d).reshape(NW, EPW)

    t = t0
    part = None
    for it in range(DEPTH_G):
        part = _sc_pass(t.reshape(4 * N_NODES, D), eidx_w, dst3, zeros)
        if it < DEPTH_G - 1:
            t = _combine(part, p, W_h, b_h2, eb_pad, w_b, b_in2)

    return _head(part, fo, w_o1b, b_o12, W_o2, b_o22)


# speedup vs baseline: 2.1890x; 2.1890x over previous
import functools

import jax
import jax.numpy as jnp
from jax import lax
from jax.experimental import pallas as pl
from jax.experimental.pallas import tpu as pltpu
from jax.experimental.pallas import tpu_sc as plsc

N_NODES = 10000
N_EDGES = 320000
D = 128
DH = 64
D_BOND = 16
N_BOND_PAD = 8
DEPTH_G = 5

NBLK = 25
BN = N_NODES // NBLK
EROWS = N_EDGES // D

NW = 32
EPW = N_EDGES // NW
CH = 80
NCH = EPW // CH
NPAD = 10240
RPT = NPAD // 16

NBUF = 2
NSUPER = NCH // NBUF


def _eidx_body(src_ref, bt_ref, eidx_ref):
    eidx_ref[...] = src_ref[...] + bt_ref[...] * N_NODES


def _eidx(src2d, bt2d):
    return pl.pallas_call(
        _eidx_body,
        out_shape=jax.ShapeDtypeStruct((EROWS, D), jnp.int32),
    )(src2d, bt2d)


def _prep_body(fnode_ref, ea_ref, wa_ref, wo1a_ref,
               eb_ref, wb_ref, bin_ref,
               p_ref, fo_ref, t0_ref):
    fn = fnode_ref[0, 0, :]
    onehot = (fn[:, None] == lax.broadcasted_iota(jnp.int32, (BN, D), 1))
    fatom = jnp.dot(onehot.astype(jnp.float32), ea_ref[...],
                    preferred_element_type=jnp.float32)
    pblk = jnp.dot(fatom, wa_ref[...], preferred_element_type=jnp.float32)
    p_ref[...] = pblk
    fo_ref[...] = jnp.dot(fatom, wo1a_ref[...],
                          preferred_element_type=jnp.float32)
    bv = jnp.dot(eb_ref[...], wb_ref[...],
                 preferred_element_type=jnp.float32) + bin_ref[...]
    for b in range(4):
        t0_ref[b] = jnp.maximum(pblk + bv[b:b + 1, :], 0.0)


def _prep(fnode3, ea_pad, w_a, w_o1a, eb_pad, w_b, b_in2):
    full = lambda shp: pl.BlockSpec(shp, lambda i: tuple(0 for _ in shp))
    return pl.pallas_call(
        _prep_body,
        grid=(NBLK,),
        in_specs=[
            pl.BlockSpec((1, 1, BN), lambda i: (i, 0, 0)),
            full((D, D)), full((D, D)), full((D, D)),
            full((N_BOND_PAD, D_BOND)), full((D_BOND, D)), full((1, D)),
        ],
        out_specs=[
            pl.BlockSpec((BN, D), lambda i: (i, 0)),
            pl.BlockSpec((BN, D), lambda i: (i, 0)),
            pl.BlockSpec((4, BN, D), lambda i: (0, i, 0)),
        ],
        out_shape=[
            jax.ShapeDtypeStruct((N_NODES, D), jnp.float32),
            jax.ShapeDtypeStruct((N_NODES, D), jnp.float32),
            jax.ShapeDtypeStruct((4, N_NODES, D), jnp.float32),
        ],
    )(fnode3, ea_pad, w_a, w_o1a, eb_pad, w_b, b_in2)


def _combine_body(part_ref, p_ref, wh_ref, bh_ref, eb_ref, wb_ref, bin_ref,
                  t_ref):
    agg = part_ref[0] + part_ref[1]
    q = p_ref[...] + jnp.dot(agg, wh_ref[...],
                             preferred_element_type=jnp.float32) + bh_ref[...]
    bv = jnp.dot(eb_ref[...], wb_ref[...],
                 preferred_element_type=jnp.float32) + bin_ref[...]
    for b in range(4):
        t_ref[b] = jnp.maximum(q + bv[b:b + 1, :], 0.0)


def _combine(part, p, w_h, b_h2, eb_pad, w_b, b_in2):
    full = lambda shp: pl.BlockSpec(shp, lambda i: tuple(0 for _ in shp))
    return pl.pallas_call(
        _combine_body,
        grid=(NBLK,),
        in_specs=[
            pl.BlockSpec((2, BN, D), lambda i: (0, i, 0)),
            pl.BlockSpec((BN, D), lambda i: (i, 0)),
            full((D, D)), full((1, D)),
            full((N_BOND_PAD, D_BOND)), full((D_BOND, D)), full((1, D)),
        ],
        out_specs=pl.BlockSpec((4, BN, D), lambda i: (0, i, 0)),
        out_shape=jax.ShapeDtypeStruct((4, N_NODES, D), jnp.float32),
    )(part, p, w_h, b_h2, eb_pad, w_b, b_in2)


def _head_body(part_ref, fo_ref, wo1b_ref, bo1_ref, wo2_ref, bo2_ref,
               out_ref):
    a = part_ref[0] + part_ref[1]
    h = jnp.maximum(
        fo_ref[...] + jnp.dot(a, wo1b_ref[...],
                              preferred_element_type=jnp.float32)
        + bo1_ref[...], 0.0)
    out_ref[...] = jnp.dot(h, wo2_ref[...],
                           preferred_element_type=jnp.float32) + bo2_ref[...]


def _head(part, fo, w_o1b, b_o12, w_o2, b_o22):
    full = lambda shp: pl.BlockSpec(shp, lambda i: tuple(0 for _ in shp))
    return pl.pallas_call(
        _head_body,
        grid=(NBLK,),
        in_specs=[
            pl.BlockSpec((2, BN, D), lambda i: (0, i, 0)),
            pl.BlockSpec((BN, D), lambda i: (i, 0)),
            full((D, D)), full((1, D)), full((D, D)), full((1, D)),
        ],
        out_specs=pl.BlockSpec((BN, D), lambda i: (i, 0)),
        out_shape=jax.ShapeDtypeStruct((N_NODES, D), jnp.float32),
    )(part, fo, w_o1b, b_o12, w_o2, b_o22)


def _sc_body(t_hbm, eidx_hbm, dst_hbm, zeros_hbm, out_hbm,
             agg_s, eidx_v, dst_v, rows_v, *sems):
    sg = sems[:NBUF]
    ss_a, ss_b = sems[NBUF], sems[NBUF + 1]
    cid = lax.axis_index("c")
    sid = lax.axis_index("s")
    w = sid * 2 + cid

    def fire_gather(c, b):
        base = pl.multiple_of(c * CH, 8)
        pltpu.async_copy(t_hbm.at[eidx_v.at[pl.ds(base, CH)]],
                         rows_v.at[b], sg[b])

    def wait_gather(b):
        pltpu.make_async_copy(t_hbm.at[pl.ds(0, CH)], rows_v.at[b],
                              sg[b]).wait()

    def fire_scatter(c, b, ss):
        pltpu.async_copy(rows_v.at[b], agg_s.at[dst_v.at[c]], ss, add=True)

    def drain(ss):
        pltpu.make_async_copy(t_hbm.at[pl.ds(0, CH)], rows_v.at[0],
                              ss).wait()

    pltpu.sync_copy(zeros_hbm, agg_s.at[pl.ds(sid * RPT, RPT)])
    pltpu.sync_copy(eidx_hbm.at[w], eidx_v)
    pltpu.sync_copy(dst_hbm.at[w], dst_v)
    plsc.subcore_barrier()

    fire_gather(0, 0)

    def super_block(s_idx, carry):
        c0 = s_idx * NBUF

        @pl.when(s_idx > 0)
        def _():
            drain(ss_b)
        fire_gather(c0 + 1, 1)
        wait_gather(0)
        fire_scatter(c0, 0, ss_a)

        @pl.when(s_idx < NSUPER - 1)
        def _():
            drain(ss_a)
            fire_gather(c0 + 2, 0)
        wait_gather(1)
        fire_scatter(c0 + 1, 1, ss_b)
        return carry

    lax.fori_loop(0, NSUPER, super_block, 0)

    drain(ss_a)
    fire_gather(NCH - 1, 0)
    drain(ss_b)
    wait_gather(0)
    fire_scatter(NCH - 1, 0, ss_a)
    drain(ss_a)

    plsc.subcore_barrier()
    pltpu.sync_copy(agg_s.at[pl.ds(sid * RPT, RPT)],
                    out_hbm.at[cid, pl.ds(sid * RPT, RPT)])


_sc_pass = functools.partial(
    pl.kernel,
    out_type=jax.ShapeDtypeStruct((2, NPAD, D), jnp.float32),
    mesh=plsc.VectorSubcoreMesh(core_axis_name="c", subcore_axis_name="s"),
    scratch_types=[
        pltpu.VMEM_SHARED((NPAD, D), jnp.float32),
        pltpu.VMEM((EPW,), jnp.int32),
        pltpu.VMEM((NCH, CH), jnp.int32),
        pltpu.VMEM((NBUF, CH, D), jnp.float32),
    ] + [pltpu.SemaphoreType.DMA] * (NBUF + 2),
)(_sc_body)


def kernel(fnode, edge_index, bond_type, E_a, E_b,
           W_in, b_in, W_h, b_h, W_o1, b_o1, W_o2, b_o2):
    fnode3 = fnode.astype(jnp.int32).reshape(NBLK, 1, BN)
    src2d = edge_index[0].astype(jnp.int32).reshape(EROWS, D)
    bt2d = bond_type.astype(jnp.int32).reshape(EROWS, D)
    dst3 = edge_index[1].astype(jnp.int32).reshape(NW, NCH, CH)

    ea_pad = jnp.zeros((D, D), jnp.float32).at[:E_a.shape[0]].set(E_a)
    eb_pad = jnp.zeros((N_BOND_PAD, D_BOND), jnp.float32).at[:4].set(E_b)
    w_a, w_b = W_in[:D], W_in[D:]
    w_o1a, w_o1b = W_o1[:D], W_o1[D:]
    b_in2 = b_in.reshape(1, D)
    b_h2 = b_h.reshape(1, D)
    b_o12 = b_o1.reshape(1, D)
    b_o22 = b_o2.reshape(1, D)
    zeros = jnp.zeros((RPT, D), jnp.float32)

    p, fo, t0 = _prep(fnode3, ea_pad, w_a, w_o1a, eb_pad, w_b, b_in2)
    eidx_w = _eidx(src2d, bt2d).reshape(NW, EPW)

    t = t0
    part = None
    for it in range(DEPTH_G):
        part = _sc_pass(t.reshape(4 * N_NODES, D), eidx_w, dst3, zeros)
        if it < DEPTH_G - 1:
            t = _combine(part, p, W_h, b_h2, eb_pad, w_b, b_in2)

    return _head(part, fo, w_o1b, b_o12, W_o2, b_o22)


# speedup vs baseline: 2.1900x; 1.0004x over previous
import functools

import jax
import jax.numpy as jnp
from jax import lax
from jax.experimental import pallas as pl
from jax.experimental.pallas import tpu as pltpu
from jax.experimental.pallas import tpu_sc as plsc

N_NODES = 10000
N_EDGES = 320000
D = 128
D_BOND = 16
N_BOND_PAD = 8
DEPTH_G = 5

NBLK = 25
BN = N_NODES // NBLK
EROWS = N_EDGES // D

NW = 32
EPW = N_EDGES // NW
CH = 80
NCH = EPW // CH
NPAD = 10240
RPT = NPAD // 16

NBUF = 2
NSUPER = NCH // NBUF


def _eidx_body(src_ref, bt_ref, eidx_ref):
    eidx_ref[...] = src_ref[...] + bt_ref[...] * N_NODES


def _eidx(src2d, bt2d):
    return pl.pallas_call(
        _eidx_body,
        out_shape=jax.ShapeDtypeStruct((EROWS, D), jnp.int32),
    )(src2d, bt2d)


def _prep_body(fnode_ref, ea_ref, wa_ref, wo1a_ref,
               eb_ref, wb_ref, bin_ref,
               p_ref, fo_ref, t0_ref):
    fn = fnode_ref[0, 0, :]
    onehot = (fn[:, None] == lax.broadcasted_iota(jnp.int32, (BN, D), 1))
    fatom = jnp.dot(onehot.astype(jnp.float32), ea_ref[...],
                    preferred_element_type=jnp.float32)
    pblk = jnp.dot(fatom, wa_ref[...], preferred_element_type=jnp.float32)
    p_ref[...] = pblk
    fo_ref[...] = jnp.dot(fatom, wo1a_ref[...],
                          preferred_element_type=jnp.float32)
    bv = jnp.dot(eb_ref[...], wb_ref[...],
                 preferred_element_type=jnp.float32) + bin_ref[...]
    for b in range(4):
        t0_ref[b] = jnp.maximum(pblk + bv[b:b + 1, :], 0.0)


def _prep(fnode3, ea_pad, w_a, w_o1a, eb_pad, w_b, b_in2):
    full = lambda shp: pl.BlockSpec(shp, lambda i: tuple(0 for _ in shp))
    return pl.pallas_call(
        _prep_body,
        grid=(NBLK,),
        in_specs=[
            pl.BlockSpec((1, 1, BN), lambda i: (i, 0, 0)),
            full((D, D)), full((D, D)), full((D, D)),
            full((N_BOND_PAD, D_BOND)), full((D_BOND, D)), full((1, D)),
        ],
        out_specs=[
            pl.BlockSpec((BN, D), lambda i: (i, 0)),
            pl.BlockSpec((BN, D), lambda i: (i, 0)),
            pl.BlockSpec((4, BN, D), lambda i: (0, i, 0)),
        ],
        out_shape=[
            jax.ShapeDtypeStruct((N_NODES, D), jnp.float32),
            jax.ShapeDtypeStruct((N_NODES, D), jnp.float32),
            jax.ShapeDtypeStruct((4, N_NODES, D), jnp.float32),
        ],
    )(fnode3, ea_pad, w_a, w_o1a, eb_pad, w_b, b_in2)


def _combine_body(part_ref, p_ref, wh_ref, bh_ref, eb_ref, wb_ref, bin_ref,
                  t_ref):
    agg = part_ref[0] + part_ref[1]
    q = p_ref[...] + jnp.dot(agg, wh_ref[...],
                             preferred_element_type=jnp.float32) + bh_ref[...]
    bv = jnp.dot(eb_ref[...], wb_ref[...],
                 preferred_element_type=jnp.float32) + bin_ref[...]
    for b in range(4):
        t_ref[b] = jnp.maximum(q + bv[b:b + 1, :], 0.0)


def _combine(part, p, w_h, b_h2, eb_pad, w_b, b_in2):
    full = lambda shp: pl.BlockSpec(shp, lambda i: tuple(0 for _ in shp))
    return pl.pallas_call(
        _combine_body,
        grid=(NBLK,),
        in_specs=[
            pl.BlockSpec((2, BN, D), lambda i: (0, i, 0)),
            pl.BlockSpec((BN, D), lambda i: (i, 0)),
            full((D, D)), full((1, D)),
            full((N_BOND_PAD, D_BOND)), full((D_BOND, D)), full((1, D)),
        ],
        out_specs=pl.BlockSpec((4, BN, D), lambda i: (0, i, 0)),
        out_shape=jax.ShapeDtypeStruct((4, N_NODES, D), jnp.float32),
    )(part, p, w_h, b_h2, eb_pad, w_b, b_in2)


def _head_body(part_ref, fo_ref, wo1b_ref, bo1_ref, wo2_ref, bo2_ref,
               out_ref):
    a = part_ref[0] + part_ref[1]
    h = jnp.maximum(
        fo_ref[...] + jnp.dot(a, wo1b_ref[...],
                              preferred_element_type=jnp.float32)
        + bo1_ref[...], 0.0)
    out_ref[...] = jnp.dot(h, wo2_ref[...],
                           preferred_element_type=jnp.float32) + bo2_ref[...]


def _head(part, fo, w_o1b, b_o12, w_o2, b_o22):
    full = lambda shp: pl.BlockSpec(shp, lambda i: tuple(0 for _ in shp))
    return pl.pallas_call(
        _head_body,
        grid=(NBLK,),
        in_specs=[
            pl.BlockSpec((2, BN, D), lambda i: (0, i, 0)),
            pl.BlockSpec((BN, D), lambda i: (i, 0)),
            full((D, D)), full((1, D)), full((D, D)), full((1, D)),
        ],
        out_specs=pl.BlockSpec((BN, D), lambda i: (i, 0)),
        out_shape=jax.ShapeDtypeStruct((N_NODES, D), jnp.float32),
    )(part, fo, w_o1b, b_o12, w_o2, b_o22)


def _sc_body(t_hbm, eidx_hbm, dst_hbm, zeros_hbm, out_hbm,
             agg_s, eidx_v, dst_v, rows_v, *sems):
    sg = sems[:NBUF]
    ss_a, ss_b = sems[NBUF], sems[NBUF + 1]
    cid = lax.axis_index("c")
    sid = lax.axis_index("s")
    w = sid * 2 + cid

    def fire_gather(c, b):
        base = pl.multiple_of(c * CH, 8)
        pltpu.async_copy(t_hbm.at[eidx_v.at[pl.ds(base, CH)]],
                         rows_v.at[b], sg[b])

    def wait_gather(b):
        pltpu.make_async_copy(t_hbm.at[pl.ds(0, CH)], rows_v.at[b],
                              sg[b]).wait()

    def fire_scatter(c, b, ss):
        pltpu.async_copy(rows_v.at[b], agg_s.at[dst_v.at[c]], ss, add=True)

    def drain(ss):
        pltpu.make_async_copy(t_hbm.at[pl.ds(0, CH)], rows_v.at[0],
                              ss).wait()

    pltpu.sync_copy(zeros_hbm, agg_s.at[pl.ds(sid * RPT, RPT)])
    pltpu.sync_copy(eidx_hbm.at[w], eidx_v)
    pltpu.sync_copy(dst_hbm.at[w], dst_v)
    plsc.subcore_barrier()

    fire_gather(0, 0)

    def super_block(s_idx, carry):
        c0 = s_idx * NBUF

        @pl.when(s_idx > 0)
        def _():
            drain(ss_b)
        fire_gather(c0 + 1, 1)
        wait_gather(0)
        fire_scatter(c0, 0, ss_a)

        @pl.when(s_idx < NSUPER - 1)
        def _():
            drain(ss_a)
            fire_gather(c0 + 2, 0)
        wait_gather(1)
        fire_scatter(c0 + 1, 1, ss_b)
        return carry

    lax.fori_loop(0, NSUPER, super_block, 0)

    drain(ss_a)
    fire_gather(NCH - 1, 0)
    drain(ss_b)
    wait_gather(0)
    fire_scatter(NCH - 1, 0, ss_a)
    drain(ss_a)

    plsc.subcore_barrier()
    pltpu.sync_copy(agg_s.at[pl.ds(sid * RPT, RPT)],
                    out_hbm.at[cid, pl.ds(sid * RPT, RPT)])


_sc_pass = functools.partial(
    pl.kernel,
    out_type=jax.ShapeDtypeStruct((2, NPAD, D), jnp.float32),
    mesh=plsc.VectorSubcoreMesh(core_axis_name="c", subcore_axis_name="s"),
    scratch_types=[
        pltpu.VMEM_SHARED((NPAD, D), jnp.float32),
        pltpu.VMEM((EPW,), jnp.int32),
        pltpu.VMEM((NCH, CH), jnp.int32),
        pltpu.VMEM((NBUF, CH, D), jnp.float32),
    ] + [pltpu.SemaphoreType.DMA] * (NBUF + 2),
)(_sc_body)


def kernel(fnode, edge_index, bond_type, E_a, E_b,
           W_in, b_in, W_h, b_h, W_o1, b_o1, W_o2, b_o2):
    fnode3 = fnode.astype(jnp.int32).reshape(NBLK, 1, BN)
    src2d = edge_index[0].astype(jnp.int32).reshape(EROWS, D)
    bt2d = bond_type.astype(jnp.int32).reshape(EROWS, D)
    dst3 = edge_index[1].astype(jnp.int32).reshape(NW, NCH, CH)

    ea_pad = jnp.zeros((D, D), jnp.float32).at[:E_a.shape[0]].set(E_a)
    eb_pad = jnp.zeros((N_BOND_PAD, D_BOND), jnp.float32).at[:4].set(E_b)
    w_a, w_b = W_in[:D], W_in[D:]
    w_o1a, w_o1b = W_o1[:D], W_o1[D:]
    b_in2 = b_in.reshape(1, D)
    b_h2 = b_h.reshape(1, D)
    b_o12 = b_o1.reshape(1, D)
    b_o22 = b_o2.reshape(1, D)
    zeros = jnp.zeros((RPT, D), jnp.float32)

    p, fo, t0 = _prep(fnode3, ea_pad, w_a, w_o1a, eb_pad, w_b, b_in2)
    eidx_w = _eidx(src2d, bt2d).reshape(NW, EPW)

    t = t0
    part = None
    for it in range(DEPTH_G):
        part = _sc_pass(t.reshape(4 * N_NODES, D), eidx_w, dst3, zeros)
        if it < DEPTH_G - 1:
            t = _combine(part, p, W_h, b_h2, eb_pad, w_b, b_in2)

    return _head(part, fo, w_o1b, b_o12, W_o2, b_o22)
